# baseline (speedup 1.0000x reference)
import jax
import jax.numpy as jnp
from jax.experimental import pallas as pl
from jax.experimental.pallas import tpu as pltpu

_EOS = 2
_CR = 8


def _eos_kernel(ids_hbm, scores_hbm, out_hbm, ids_ref, buf_ref,
                in_sems, out_sems, ids_sem):
    rows = ids_ref.shape[0]
    n_chunks = rows // _CR

    def in_cp(c):
        return pltpu.make_async_copy(
            scores_hbm.at[pl.ds(c * _CR, _CR), :],
            buf_ref.at[pl.ds(c * _CR, _CR), :],
            in_sems.at[c])

    def out_cp(c):
        return pltpu.make_async_copy(
            buf_ref.at[pl.ds(c * _CR, _CR), :],
            out_hbm.at[pl.ds(c * _CR, _CR), :],
            out_sems.at[c])

    for c in range(n_chunks):
        in_cp(c).start()
    ids_cp = pltpu.make_async_copy(ids_hbm, ids_ref, ids_sem)
    ids_cp.start()
    ids_cp.wait()

    counts = jnp.sum((ids_ref[...] == _EOS).astype(jnp.int32), axis=1,
                     keepdims=True)
    count_init = counts
    done = (counts - count_init) >= 2
    n_done = jnp.sum(done.astype(jnp.int32))

    @pl.when(n_done == 0)
    def _fast():
        for c in range(n_chunks):
            in_cp(c).wait()
            out_cp(c).start()
        for c in range(n_chunks):
            out_cp(c).wait()

    @pl.when(n_done != 0)
    def _masked():
        for c in range(n_chunks):
            in_cp(c).wait()
        block = buf_ref[...]
        masked = jnp.where(done, -jnp.inf, block)
        buf_ref[...] = masked
        buf_ref[:, _EOS:_EOS + 1] = jnp.where(
            done, 0.0, block[:, _EOS:_EOS + 1])
        for c in range(n_chunks):
            out_cp(c).start()
        for c in range(n_chunks):
            out_cp(c).wait()


def kernel(input_ids, scores):
    batch, vocab = scores.shape
    return pl.pallas_call(
        _eos_kernel,
        in_specs=[
            pl.BlockSpec(memory_space=pl.ANY),
            pl.BlockSpec(memory_space=pl.ANY),
        ],
        out_specs=pl.BlockSpec(memory_space=pl.ANY),
        out_shape=jax.ShapeDtypeStruct(scores.shape, scores.dtype),
        scratch_shapes=[
            pltpu.VMEM(input_ids.shape, input_ids.dtype),
            pltpu.VMEM((batch, vocab), jnp.float32),
            pltpu.SemaphoreType.DMA((batch // _CR,)),
            pltpu.SemaphoreType.DMA((batch // _CR,)),
            pltpu.SemaphoreType.DMA,
        ],
    )(input_ids, scores)
